# Initial kernel scaffold; baseline (speedup 1.0000x reference)
import jax
import jax.numpy as jnp
from jax.experimental import pallas as pl


def kernel(x0, edge_index, batch, W1, b1, W2, b2, W3, b3, Wl1, bl1, Wl2, bl2):
    raise NotImplementedError("write your pallas kernel here")


# speedup vs baseline: 23.1671x; 23.1671x over previous
import functools

import jax
import jax.numpy as jnp
from jax import lax
from jax.experimental import pallas as pl
from jax.experimental.pallas import tpu as pltpu
from jax.experimental.pallas import tpu_sc as plsc

_N = 10000
_E = 320000
_D = 128
_H = 32
_C = 10

_NC = 2
_NS = 16
_NW = _NC * _NS

_EPW = _E // _NW
_K = 80
_NCHUNK = _EPW // _K
_N2 = 10240
_RPS2 = _N2 // _NS

_f32 = jnp.float32

_sc_mesh = plsc.VectorSubcoreMesh(core_axis_name="c", subcore_axis_name="s")
_sc_params = pltpu.CompilerParams(use_tc_tiling_on_sc=False)


@functools.partial(
    pl.kernel,
    mesh=_sc_mesh,
    out_type=jax.ShapeDtypeStruct((2 * _N2,), _f32),
    scratch_types=[
        pltpu.VMEM((_NCHUNK, _K), jnp.int32),
        pltpu.VMEM((_K,), _f32),
        pltpu.VMEM_SHARED((_N2,), _f32),
    ],
    compiler_params=_sc_params,
)
def _sc_deg(dst_hbm, zeros1_hbm, out_hbm, didx, ones_v, acc):
    c = lax.axis_index("c")
    s = lax.axis_index("s")
    w = s * _NC + c

    pltpu.sync_copy(dst_hbm.at[w], didx)
    for j in range(_K // 16):
        ones_v[pl.ds(j * 16, 16)] = jnp.ones((16,), _f32)
    pltpu.sync_copy(zeros1_hbm.at[pl.ds(s * _RPS2, _RPS2)],
                    acc.at[pl.ds(s * _RPS2, _RPS2)])
    plsc.subcore_barrier()

    def body(i, carry):
        pltpu.sync_copy(ones_v, acc.at[didx.at[i]], add=True)
        return carry

    lax.fori_loop(0, _NCHUNK, body, 0)
    plsc.subcore_barrier()
    pltpu.sync_copy(acc.at[pl.ds(s * _RPS2, _RPS2)],
                    out_hbm.at[pl.ds(c * _N2 + s * _RPS2, _RPS2)])


@functools.partial(
    pl.kernel,
    mesh=_sc_mesh,
    out_type=jax.ShapeDtypeStruct((2 * _N2, _H), _f32),
    scratch_types=[
        pltpu.VMEM((_NCHUNK, _K), jnp.int32),
        pltpu.VMEM((_NCHUNK, _K), jnp.int32),
        pltpu.VMEM((_K, _H), _f32),
        pltpu.VMEM_SHARED((_N2, _H), _f32),
        pltpu.SemaphoreType.DMA,
    ],
    compiler_params=_sc_params,
)
def _sc_scatter(g_hbm, src_hbm, dst_hbm, zeros2_hbm, out_hbm,
                sidx, didx, rows, acc, sem):
    c = lax.axis_index("c")
    s = lax.axis_index("s")
    w = s * _NC + c

    pltpu.sync_copy(src_hbm.at[w], sidx)
    pltpu.sync_copy(dst_hbm.at[w], didx)
    pltpu.sync_copy(zeros2_hbm.at[pl.ds(s * _RPS2, _RPS2)],
                    acc.at[pl.ds(s * _RPS2, _RPS2)])
    plsc.subcore_barrier()

    def body(i, carry):
        pltpu.async_copy(g_hbm.at[sidx.at[i]], rows, sem).wait()
        pltpu.sync_copy(rows, acc.at[didx.at[i]], add=True)
        return carry

    lax.fori_loop(0, _NCHUNK, body, 0)
    plsc.subcore_barrier()
    pltpu.sync_copy(acc.at[pl.ds(s * _RPS2, _RPS2)],
                    out_hbm.at[pl.ds(c * _N2 + s * _RPS2, _RPS2)])


def _tc_first_body(x0_ref, w1_ref, degp_ref, g1_ref, dinv_ref):
    deg = 1.0 + degp_ref[0] + degp_ref[1]
    dinv = lax.rsqrt(deg)
    dinv32 = jnp.broadcast_to(dinv, (_N, _H))
    h1 = jnp.dot(x0_ref[...], w1_ref[...], preferred_element_type=_f32)
    g1_ref[...] = h1 * dinv32
    dinv_ref[...] = dinv32


def _tc_mid_body(p_ref, g_ref, dinv_ref, b_ref, w_ref, gn_ref):
    dinv = dinv_ref[...]
    x = jnp.maximum(dinv * (p_ref[0] + p_ref[1] + g_ref[...]) + b_ref[...], 0.0)
    gn_ref[...] = jnp.dot(x, w_ref[...], preferred_element_type=_f32) * dinv


def _tc_head_body(p_ref, g_ref, dinv_ref, b3_ref, wl1_ref, bl1_ref,
                  wl2_ref, bl2_ref, out_ref):
    dinv = dinv_ref[...]
    x3 = jnp.maximum(dinv * (p_ref[0] + p_ref[1] + g_ref[...]) + b3_ref[...], 0.0)
    h = jnp.maximum(
        jnp.dot(x3, wl1_ref[...], preferred_element_type=_f32) + bl1_ref[...], 0.0)
    out_ref[...] = (
        jnp.dot(h, wl2_ref[...], preferred_element_type=_f32) + bl2_ref[...])


def kernel(x0, edge_index, batch, W1, b1, W2, b2, W3, b3, Wl1, bl1, Wl2, bl2):
    src = edge_index[0].reshape(_NW, _NCHUNK, _K)
    dst = edge_index[1].reshape(_NW, _NCHUNK, _K)
    zeros1 = jnp.zeros((_N2,), _f32)
    zeros2 = jnp.zeros((_N2, _H), _f32)

    degp = _sc_deg(dst, zeros1)
    degp3 = degp.reshape(2, _N2)[:, :_N].reshape(2, _N, 1)

    g1, dinv32 = pl.pallas_call(
        _tc_first_body,
        out_shape=[jax.ShapeDtypeStruct((_N, _H), _f32),
                   jax.ShapeDtypeStruct((_N, _H), _f32)],
    )(x0, W1, degp3)

    p1 = _sc_scatter(g1, src, dst, zeros2).reshape(2, _N2, _H)[:, :_N, :]
    g2 = pl.pallas_call(
        _tc_mid_body,
        out_shape=jax.ShapeDtypeStruct((_N, _H), _f32),
    )(p1, g1, dinv32, b1.reshape(1, _H), W2)

    p2 = _sc_scatter(g2, src, dst, zeros2).reshape(2, _N2, _H)[:, :_N, :]
    g3 = pl.pallas_call(
        _tc_mid_body,
        out_shape=jax.ShapeDtypeStruct((_N, _H), _f32),
    )(p2, g2, dinv32, b2.reshape(1, _H), W3)

    p3 = _sc_scatter(g3, src, dst, zeros2).reshape(2, _N2, _H)[:, :_N, :]
    out = pl.pallas_call(
        _tc_head_body,
        out_shape=jax.ShapeDtypeStruct((_N, _C), _f32),
    )(p3, g3, dinv32, b3.reshape(1, _H), Wl1, bl1.reshape(1, 16),
      Wl2, bl2.reshape(1, _C))
    return out


# speedup vs baseline: 46.3116x; 1.9990x over previous
import functools

import jax
import jax.numpy as jnp
from jax import lax
from jax.experimental import pallas as pl
from jax.experimental.pallas import tpu as pltpu
from jax.experimental.pallas import tpu_sc as plsc

_N = 10000
_E = 320000
_D = 128
_H = 32
_C = 10

_NC = 2
_NS = 16
_NW = _NC * _NS

_EPW = _E // _NW
_K = 100
_NCHUNK = _EPW // _K
_G = 5
_NG = _NCHUNK // _G
_N2 = 10240
_RPS2 = _N2 // _NS

_f32 = jnp.float32

_sc_mesh = plsc.VectorSubcoreMesh(core_axis_name="c", subcore_axis_name="s")
_sc_params = pltpu.CompilerParams(use_tc_tiling_on_sc=False)


@functools.partial(
    pl.kernel,
    mesh=_sc_mesh,
    out_type=jax.ShapeDtypeStruct((2 * _N2,), _f32),
    scratch_types=[
        pltpu.VMEM((_NCHUNK, _K), jnp.int32),
        pltpu.VMEM((_K,), _f32),
        pltpu.VMEM_SHARED((_N2,), _f32),
    ],
    compiler_params=_sc_params,
)
def _sc_deg(dst_hbm, zeros1_hbm, out_hbm, didx, ones_v, acc):
    c = lax.axis_index("c")
    s = lax.axis_index("s")
    w = s * _NC + c

    pltpu.sync_copy(dst_hbm.at[w], didx)
    for j in range(_K // 16):
        ones_v[pl.ds(j * 16, 16)] = jnp.ones((16,), _f32)
    pltpu.sync_copy(zeros1_hbm.at[pl.ds(s * _RPS2, _RPS2)],
                    acc.at[pl.ds(s * _RPS2, _RPS2)])
    plsc.subcore_barrier()

    def body(i, carry):
        pltpu.sync_copy(ones_v, acc.at[didx.at[i]], add=True)
        return carry

    lax.fori_loop(0, _NCHUNK, body, 0)
    plsc.subcore_barrier()
    pltpu.sync_copy(acc.at[pl.ds(s * _RPS2, _RPS2)],
                    out_hbm.at[pl.ds(c * _N2 + s * _RPS2, _RPS2)])


@functools.partial(
    pl.kernel,
    mesh=_sc_mesh,
    out_type=jax.ShapeDtypeStruct((2 * _N2, _H), _f32),
    scratch_types=[
        pltpu.VMEM((_NCHUNK, _K), jnp.int32),
        pltpu.VMEM((_NCHUNK, _K), jnp.int32),
        pltpu.VMEM((2, _G * _K, _H), _f32),
        pltpu.VMEM_SHARED((_N2, _H), _f32),
        pltpu.SemaphoreType.DMA,
        pltpu.SemaphoreType.DMA,
    ],
    compiler_params=_sc_params,
)
def _sc_scatter(g_hbm, src_hbm, dst_hbm, zeros2_hbm, out_hbm,
                sidx, didx, rows, acc, sem0, sem1):
    c = lax.axis_index("c")
    s = lax.axis_index("s")
    w = s * _NC + c

    pltpu.sync_copy(src_hbm.at[w], sidx)
    pltpu.sync_copy(dst_hbm.at[w], didx)
    pltpu.sync_copy(zeros2_hbm.at[pl.ds(s * _RPS2, _RPS2)],
                    acc.at[pl.ds(s * _RPS2, _RPS2)])
    plsc.subcore_barrier()

    def fire(grp, slot, sem):
        for j in range(_G):
            pltpu.async_copy(g_hbm.at[sidx.at[grp * _G + j]],
                             rows.at[slot, pl.ds(j * _K, _K)], sem)

    def drain_scatter(grp, slot, sem):
        for j in range(_G):
            pltpu.make_async_copy(zeros2_hbm.at[pl.ds(0, _K)],
                                  rows.at[slot, pl.ds(j * _K, _K)], sem).wait()
            pltpu.sync_copy(rows.at[slot, pl.ds(j * _K, _K)],
                            acc.at[didx.at[grp * _G + j]], add=True)

    fire(0, 0, sem0)

    def body(k, carry):
        g0 = 2 * k
        fire(g0 + 1, 1, sem1)
        drain_scatter(g0, 0, sem0)

        @pl.when(k < _NG // 2 - 1)
        def _():
            fire(g0 + 2, 0, sem0)

        drain_scatter(g0 + 1, 1, sem1)
        return carry

    lax.fori_loop(0, _NG // 2, body, 0)
    plsc.subcore_barrier()
    pltpu.sync_copy(acc.at[pl.ds(s * _RPS2, _RPS2)],
                    out_hbm.at[pl.ds(c * _N2 + s * _RPS2, _RPS2)])


def _tc_first_body(x0_ref, w1_ref, degp_ref, g1_ref, dinv_ref):
    deg = 1.0 + degp_ref[0] + degp_ref[1]
    dinv = lax.rsqrt(deg)
    dinv32 = jnp.broadcast_to(dinv, (_N, _H))
    h1 = jnp.dot(x0_ref[...], w1_ref[...], preferred_element_type=_f32)
    g1_ref[...] = h1 * dinv32
    dinv_ref[...] = dinv32


def _tc_mid_body(p_ref, g_ref, dinv_ref, b_ref, w_ref, gn_ref):
    dinv = dinv_ref[...]
    x = jnp.maximum(dinv * (p_ref[0] + p_ref[1] + g_ref[...]) + b_ref[...], 0.0)
    gn_ref[...] = jnp.dot(x, w_ref[...], preferred_element_type=_f32) * dinv


def _tc_head_body(p_ref, g_ref, dinv_ref, b3_ref, wl1_ref, bl1_ref,
                  wl2_ref, bl2_ref, out_ref):
    dinv = dinv_ref[...]
    x3 = jnp.maximum(dinv * (p_ref[0] + p_ref[1] + g_ref[...]) + b3_ref[...], 0.0)
    h = jnp.maximum(
        jnp.dot(x3, wl1_ref[...], preferred_element_type=_f32) + bl1_ref[...], 0.0)
    out_ref[...] = (
        jnp.dot(h, wl2_ref[...], preferred_element_type=_f32) + bl2_ref[...])


def kernel(x0, edge_index, batch, W1, b1, W2, b2, W3, b3, Wl1, bl1, Wl2, bl2):
    src = edge_index[0].reshape(_NW, _NCHUNK, _K)
    dst = edge_index[1].reshape(_NW, _NCHUNK, _K)
    zeros1 = jnp.zeros((_N2,), _f32)
    zeros2 = jnp.zeros((_N2, _H), _f32)

    degp = _sc_deg(dst, zeros1)
    degp3 = degp.reshape(2, _N2)[:, :_N].reshape(2, _N, 1)

    g1, dinv32 = pl.pallas_call(
        _tc_first_body,
        out_shape=[jax.ShapeDtypeStruct((_N, _H), _f32),
                   jax.ShapeDtypeStruct((_N, _H), _f32)],
    )(x0, W1, degp3)

    p1 = _sc_scatter(g1, src, dst, zeros2).reshape(2, _N2, _H)[:, :_N, :]
    g2 = pl.pallas_call(
        _tc_mid_body,
        out_shape=jax.ShapeDtypeStruct((_N, _H), _f32),
    )(p1, g1, dinv32, b1.reshape(1, _H), W2)

    p2 = _sc_scatter(g2, src, dst, zeros2).reshape(2, _N2, _H)[:, :_N, :]
    g3 = pl.pallas_call(
        _tc_mid_body,
        out_shape=jax.ShapeDtypeStruct((_N, _H), _f32),
    )(p2, g2, dinv32, b2.reshape(1, _H), W3)

    p3 = _sc_scatter(g3, src, dst, zeros2).reshape(2, _N2, _H)[:, :_N, :]
    out = pl.pallas_call(
        _tc_head_body,
        out_shape=jax.ShapeDtypeStruct((_N, _C), _f32),
    )(p3, g3, dinv32, b3.reshape(1, _H), Wl1, bl1.reshape(1, 16),
      Wl2, bl2.reshape(1, _C))
    return out
